# baseline (speedup 1.0000x reference)
import jax
import jax.numpy as jnp
from jax.experimental import pallas as pl
from jax.experimental.pallas import tpu as pltpu

_TOP_P = 0.8


def _routing_body(xb_ref, rw_ref, rb_ref, gates_ref, *, n_exp):
    logits = jnp.dot(xb_ref[...], rw_ref[...],
                     preferred_element_type=jnp.float32)
    logits = logits + rb_ref[...]
    m = jnp.max(logits, axis=-1, keepdims=True)
    ex = jnp.exp(logits - m)
    p = ex / jnp.sum(ex, axis=-1, keepdims=True)
    lane = jax.lax.broadcasted_iota(jnp.int32, p.shape, 1)
    rcols = []
    for i in range(n_exp):
        pi = p[:, i:i + 1]
        ahead = (p > pi) | ((p == pi) & (lane < i))
        rcols.append(jnp.sum(ahead.astype(jnp.int32), axis=-1,
                             keepdims=True))
    rank = jnp.concatenate(rcols, axis=-1)
    scols = []
    for j in range(n_exp):
        sel = (rank == j).astype(p.dtype)
        scols.append(jnp.sum(p * sel, axis=-1, keepdims=True))
    acols = [jnp.ones_like(scols[0])]
    cum = scols[0]
    for j in range(1, n_exp):
        cum = cum + scols[j]
        acols.append((cum < _TOP_P).astype(p.dtype))
    active = jnp.concatenate(acols, axis=-1)
    act_val = p * active
    for i in range(n_exp):
        sel = (rank[:, i:i + 1] == lane).astype(p.dtype)
        gates_ref[i] = jnp.sum(act_val * sel, axis=-1, keepdims=True)


def _ffn_body(xb_ref, g_ref, w1_ref, b1_ref, w2_ref, b2_ref, out_ref, *,
              n_chunks):
    e = pl.program_id(0)
    g = g_ref[0]
    n = out_ref.shape[0]
    ck = n // n_chunks
    keep = e > 0
    for s in range(n_chunks):
        lo, hi = s * ck, (s + 1) * ck
        gs = g[lo:hi]
        h = jnp.dot(xb_ref[lo:hi, :], w1_ref[0],
                    preferred_element_type=jnp.float32)
        h = jnp.maximum(h + b1_ref[0], 0.0).astype(jnp.bfloat16)
        contrib = jnp.dot(h, w2_ref[0],
                          preferred_element_type=jnp.float32)
        contrib = (contrib + b2_ref[0]) * gs
        prev = jnp.where(keep, out_ref[lo:hi, :], 0.0)
        out_ref[lo:hi, :] = prev + contrib


def kernel(x, router_w, router_b, w1, b1, w2, b2):
    B, T, H = x.shape
    E, _, I = w1.shape
    BT = B * T
    R = 256 if BT % 256 == 0 else BT
    n_chunks = BT // 512 if BT % 512 == 0 else 1

    xb = x.reshape(BT, H).astype(jnp.bfloat16)
    rwb = router_w.astype(jnp.bfloat16)
    w1b = w1.astype(jnp.bfloat16)
    w2b = w2.astype(jnp.bfloat16)
    rb2 = router_b.reshape(1, E)
    b1r = b1.reshape(E, 1, I)
    b2r = b2.reshape(E, 1, H)

    gates = pl.pallas_call(
        lambda *refs: _routing_body(*refs, n_exp=E),
        grid=(BT // R,),
        in_specs=[
            pl.BlockSpec((R, H), lambda t: (t, 0)),
            pl.BlockSpec((H, E), lambda t: (0, 0)),
            pl.BlockSpec((1, E), lambda t: (0, 0)),
        ],
        out_specs=pl.BlockSpec((E, R, 1), lambda t: (0, t, 0)),
        out_shape=jax.ShapeDtypeStruct((E, BT, 1), jnp.float32),
    )(xb, rwb, rb2)

    out = pl.pallas_call(
        lambda *refs: _ffn_body(*refs, n_chunks=n_chunks),
        grid=(E,),
        in_specs=[
            pl.BlockSpec((BT, H), lambda e: (0, 0)),
            pl.BlockSpec((1, BT, 1), lambda e: (e, 0, 0)),
            pl.BlockSpec((1, H, I), lambda e: (e, 0, 0)),
            pl.BlockSpec((1, 1, I), lambda e: (e, 0, 0)),
            pl.BlockSpec((1, I, H), lambda e: (e, 0, 0)),
            pl.BlockSpec((1, 1, H), lambda e: (e, 0, 0)),
        ],
        out_specs=pl.BlockSpec((BT, H), lambda e: (0, 0)),
        out_shape=jax.ShapeDtypeStruct((BT, H), jnp.float32),
    )(xb, gates, w1b, b1r, w2b, b2r)
    return out.reshape(B, T, H)


# speedup vs baseline: 1.0196x; 1.0196x over previous
import jax
import jax.numpy as jnp
from jax.experimental import pallas as pl
from jax.experimental.pallas import tpu as pltpu

_TOP_P = 0.8


def _routing_chunk(xb, rw, rb, gates_ref, lo, n_exp):
    logits = jnp.dot(xb, rw, preferred_element_type=jnp.float32) + rb
    m = jnp.max(logits, axis=-1, keepdims=True)
    ex = jnp.exp(logits - m)
    p = ex / jnp.sum(ex, axis=-1, keepdims=True)
    lane = jax.lax.broadcasted_iota(jnp.int32, p.shape, 1)
    rcols = []
    for i in range(n_exp):
        pi = p[:, i:i + 1]
        ahead = (p > pi) | ((p == pi) & (lane < i))
        rcols.append(jnp.sum(ahead.astype(jnp.int32), axis=-1,
                             keepdims=True))
    rank = jnp.concatenate(rcols, axis=-1)
    scols = []
    for j in range(n_exp):
        sel = (rank == j).astype(p.dtype)
        scols.append(jnp.sum(p * sel, axis=-1, keepdims=True))
    acols = [jnp.ones_like(scols[0])]
    cum = scols[0]
    for j in range(1, n_exp):
        cum = cum + scols[j]
        acols.append((cum < _TOP_P).astype(p.dtype))
    active = jnp.concatenate(acols, axis=-1)
    act_val = p * active
    for i in range(n_exp):
        sel = (rank[:, i:i + 1] == lane).astype(p.dtype)
        gates_ref[i, lo:lo + xb.shape[0]] = jnp.sum(
            act_val * sel, axis=-1, keepdims=True)


def _moe_body(xb_ref, rw_ref, rb_ref, w1_ref, b1_ref, w2_ref, b2_ref,
              out_ref, gates_ref, *, n_exp, n_chunks, r_chunk):
    e = pl.program_id(1)
    n = out_ref.shape[0]

    @pl.when(e == 0)
    def _route():
        rw = rw_ref[...]
        rb = rb_ref[...]
        for r in range(n // r_chunk):
            lo = r * r_chunk
            _routing_chunk(xb_ref[lo:lo + r_chunk, :], rw, rb,
                           gates_ref, lo, n_exp)

    g = gates_ref[e]
    ck = n // n_chunks
    keep = e > 0
    for s in range(n_chunks):
        lo, hi = s * ck, (s + 1) * ck
        gs = g[lo:hi]
        h = jnp.dot(xb_ref[lo:hi, :], w1_ref[0],
                    preferred_element_type=jnp.float32)
        h = jnp.maximum(h + b1_ref[0], 0.0).astype(jnp.bfloat16)
        contrib = jnp.dot(h, w2_ref[0],
                          preferred_element_type=jnp.float32)
        contrib = (contrib + b2_ref[0]) * gs
        prev = jnp.where(keep, out_ref[lo:hi, :], 0.0)
        out_ref[lo:hi, :] = prev + contrib


def kernel(x, router_w, router_b, w1, b1, w2, b2):
    B, T, H = x.shape
    E, _, I = w1.shape
    BT = B * T
    N = 2048 if BT % 2048 == 0 else BT
    n_chunks = 4 if N % (4 * 512) == 0 else 1
    r_chunk = 256 if N % 256 == 0 else N

    xb = x.reshape(BT, H).astype(jnp.bfloat16)
    rwb = router_w.astype(jnp.bfloat16)
    w1b = w1.astype(jnp.bfloat16)
    w2b = w2.astype(jnp.bfloat16)
    rb2 = router_b.reshape(1, E)
    b1r = b1.reshape(E, 1, I)
    b2r = b2.reshape(E, 1, H)

    grid = (BT // N, E)
    out = pl.pallas_call(
        lambda *refs: _moe_body(*refs, n_exp=E, n_chunks=n_chunks,
                                r_chunk=r_chunk),
        grid=grid,
        in_specs=[
            pl.BlockSpec((N, H), lambda t, e: (t, 0)),
            pl.BlockSpec((H, E), lambda t, e: (0, 0)),
            pl.BlockSpec((1, E), lambda t, e: (0, 0)),
            pl.BlockSpec((1, H, I), lambda t, e: (e, 0, 0)),
            pl.BlockSpec((1, 1, I), lambda t, e: (e, 0, 0)),
            pl.BlockSpec((1, I, H), lambda t, e: (e, 0, 0)),
            pl.BlockSpec((1, 1, H), lambda t, e: (e, 0, 0)),
        ],
        out_specs=pl.BlockSpec((N, H), lambda t, e: (t, 0)),
        out_shape=jax.ShapeDtypeStruct((BT, H), jnp.float32),
        scratch_shapes=[pltpu.VMEM((E, N, 1), jnp.float32)],
        compiler_params=pltpu.CompilerParams(
            dimension_semantics=("arbitrary", "arbitrary")),
    )(xb, rwb, rb2, w1b, b1r, w2b, b2r)
    return out.reshape(B, T, H)


# speedup vs baseline: 1.0312x; 1.0114x over previous
import jax
import jax.numpy as jnp
from jax.experimental import pallas as pl
from jax.experimental.pallas import tpu as pltpu

_TOP_P = 0.8


def _routing_chunk(xb, rw, rb, gates_ref, lo, n_exp):
    logits = jnp.dot(xb, rw, preferred_element_type=jnp.float32) + rb
    m = jnp.max(logits, axis=-1, keepdims=True)
    ex = jnp.exp(logits - m)
    p = ex / jnp.sum(ex, axis=-1, keepdims=True)
    lane = jax.lax.broadcasted_iota(jnp.int32, p.shape, 1)
    rcols = []
    for i in range(n_exp):
        pi = p[:, i:i + 1]
        ahead = (p > pi) | ((p == pi) & (lane < i))
        rcols.append(jnp.sum(ahead.astype(jnp.int32), axis=-1,
                             keepdims=True))
    rank = jnp.concatenate(rcols, axis=-1)
    scols = []
    for j in range(n_exp):
        sel = (rank == j).astype(p.dtype)
        scols.append(jnp.sum(p * sel, axis=-1, keepdims=True))
    acols = [jnp.ones_like(scols[0])]
    cum = scols[0]
    for j in range(1, n_exp):
        cum = cum + scols[j]
        acols.append((cum < _TOP_P).astype(p.dtype))
    active = jnp.concatenate(acols, axis=-1)
    act_val = p * active
    for i in range(n_exp):
        sel = (rank[:, i:i + 1] == lane).astype(p.dtype)
        gates_ref[i, lo:lo + xb.shape[0]] = jnp.sum(
            act_val * sel, axis=-1, keepdims=True)


def _moe_body(xb_ref, rw_ref, rb_ref, w1_ref, b1_ref, w2_ref, b2_ref,
              out_ref, gates_ref, *, n_exp, n_chunks, r_chunk):
    e = pl.program_id(1)
    n = out_ref.shape[0]

    @pl.when(e == 0)
    def _route():
        rw = rw_ref[...]
        rb = rb_ref[...]
        for r in range(n // r_chunk):
            lo = r * r_chunk
            _routing_chunk(xb_ref[lo:lo + r_chunk, :], rw, rb,
                           gates_ref, lo, n_exp)

    g = gates_ref[e]
    ck = n // n_chunks
    keep = e > 0
    for s in range(n_chunks):
        lo, hi = s * ck, (s + 1) * ck
        gs = g[lo:hi]
        h = jnp.dot(xb_ref[lo:hi, :], w1_ref[0],
                    preferred_element_type=jnp.float32)
        h = jnp.maximum(h + b1_ref[0], 0.0).astype(jnp.bfloat16)
        contrib = jnp.dot(h, w2_ref[0],
                          preferred_element_type=jnp.float32)
        contrib = (contrib + b2_ref[0]) * gs
        prev = jnp.where(keep, out_ref[lo:hi, :], 0.0)
        out_ref[lo:hi, :] = prev + contrib


def kernel(x, router_w, router_b, w1, b1, w2, b2):
    B, T, H = x.shape
    E, _, I = w1.shape
    BT = B * T
    N = 2048 if BT % 2048 == 0 else BT
    n_chunks = 8 if N % (8 * 256) == 0 else 1
    r_chunk = N

    xb = x.reshape(BT, H).astype(jnp.bfloat16)
    rwb = router_w.astype(jnp.bfloat16)
    w1b = w1.astype(jnp.bfloat16)
    w2b = w2.astype(jnp.bfloat16)
    rb2 = router_b.reshape(1, E)
    b1r = b1.reshape(E, 1, I)
    b2r = b2.reshape(E, 1, H)

    grid = (BT // N, E)
    out = pl.pallas_call(
        lambda *refs: _moe_body(*refs, n_exp=E, n_chunks=n_chunks,
                                r_chunk=r_chunk),
        grid=grid,
        in_specs=[
            pl.BlockSpec((N, H), lambda t, e: (t, 0)),
            pl.BlockSpec((H, E), lambda t, e: (0, 0)),
            pl.BlockSpec((1, E), lambda t, e: (0, 0)),
            pl.BlockSpec((1, H, I), lambda t, e: (e, 0, 0)),
            pl.BlockSpec((1, 1, I), lambda t, e: (e, 0, 0)),
            pl.BlockSpec((1, I, H), lambda t, e: (e, 0, 0)),
            pl.BlockSpec((1, 1, H), lambda t, e: (e, 0, 0)),
        ],
        out_specs=pl.BlockSpec((N, H), lambda t, e: (t, 0)),
        out_shape=jax.ShapeDtypeStruct((BT, H), jnp.float32),
        scratch_shapes=[pltpu.VMEM((E, N, 1), jnp.float32)],
        compiler_params=pltpu.CompilerParams(
            dimension_semantics=("arbitrary", "arbitrary")),
    )(xb, rwb, rb2, w1b, b1r, w2b, b2r)
    return out.reshape(B, T, H)
